# Initial kernel scaffold; baseline (speedup 1.0000x reference)
import jax
import jax.numpy as jnp
from jax.experimental import pallas as pl


def kernel(node_attr, edge_attr, edge_index, num_nodes, eb_W0, eb_b0, eb_W1, eb_b1, eb_W2, eb_b2, eb_W3, eb_b3, eb_g, eb_beta, nb_W0, nb_b0, nb_W1, nb_b1, nb_W2, nb_b2, nb_W3, nb_b3, nb_g, nb_beta):
    raise NotImplementedError("write your pallas kernel here")


# speedup vs baseline: 2.5493x; 2.5493x over previous
import functools

import jax
import jax.numpy as jnp
from jax import lax
from jax.experimental import pallas as pl
from jax.experimental.pallas import tpu as pltpu
from jax.experimental.pallas import tpu_sc as plsc

H = 128
GROUP = 128


def _sc_gather(node_attr, senders, receivers):
    N, Hd = node_attr.shape
    E = senders.shape[0]
    info = plsc.get_sparse_core_info()
    NC, NS = info.num_cores, info.num_subcores
    NW = NC * NS
    NG = E // GROUP
    assert NG * GROUP == E
    KMAX = (NG + NW - 1) // NW

    mesh = plsc.VectorSubcoreMesh(core_axis_name="c", subcore_axis_name="s")

    @functools.partial(
        pl.kernel,
        mesh=mesh,
        out_type=(
            jax.ShapeDtypeStruct((E, Hd), jnp.float32),
            jax.ShapeDtypeStruct((E, Hd), jnp.float32),
        ),
        scratch_types=[
            pltpu.VMEM((GROUP,), jnp.int32),
            pltpu.VMEM((GROUP,), jnp.int32),
            pltpu.VMEM((GROUP, Hd), jnp.float32),
            pltpu.VMEM((GROUP, Hd), jnp.float32),
            pltpu.SemaphoreType.DMA,
            pltpu.SemaphoreType.DMA,
        ],
    )
    def k(na_hbm, s_hbm, r_hbm, gs_hbm, gr_hbm, sidx, ridx, bufs, bufr, sem1, sem2):
        wid = lax.axis_index("s") * NC + lax.axis_index("c")

        def body(kk, carry):
            g = kk * NW + wid

            @pl.when(g < NG)
            def _():
                off = g * GROUP
                pltpu.sync_copy(s_hbm.at[pl.ds(off, GROUP)], sidx)
                pltpu.sync_copy(r_hbm.at[pl.ds(off, GROUP)], ridx)
                cs = pltpu.async_copy(na_hbm.at[sidx], bufs, sem1)
                cr = pltpu.async_copy(na_hbm.at[ridx], bufr, sem2)
                cs.wait()
                cr.wait()
                pltpu.sync_copy(bufs, gs_hbm.at[pl.ds(off, GROUP)])
                pltpu.sync_copy(bufr, gr_hbm.at[pl.ds(off, GROUP)])

            return carry

        lax.fori_loop(0, KMAX, body, 0)

    return k(node_attr, senders, receivers)


def _sc_scatter(ue, receivers, N):
    E, Hd = ue.shape
    info = plsc.get_sparse_core_info()
    NC, NS = info.num_cores, info.num_subcores
    NW = NC * NS
    NG = E // GROUP
    KMAX = (NG + NW - 1) // NW
    ZCH = 80
    NCH = N // ZCH
    assert NCH * ZCH == N
    ZROUNDS = (NCH + NS - 1) // NS

    mesh = plsc.VectorSubcoreMesh(core_axis_name="c", subcore_axis_name="s")

    @functools.partial(
        pl.kernel,
        mesh=mesh,
        out_type=jax.ShapeDtypeStruct((NC, N, Hd), jnp.float32),
        scratch_types=[
            pltpu.VMEM((GROUP,), jnp.int32),
            pltpu.VMEM((GROUP, Hd), jnp.float32),
            pltpu.VMEM_SHARED((N, Hd), jnp.float32),
        ],
    )
    def k(ue_hbm, r_hbm, out_hbm, ridx, buf, acc):
        cid = lax.axis_index("c")
        sid = lax.axis_index("s")
        wid = sid * NC + cid

        def zbody(j, carry):
            for i in range(Hd // 16):
                buf[j, pl.ds(i * 16, 16)] = jnp.zeros((16,), jnp.float32)
            return carry

        lax.fori_loop(0, GROUP, zbody, 0)
        for q in range(ZROUNDS):
            ch = sid + q * NS

            @pl.when(ch < NCH)
            def _():
                pltpu.sync_copy(buf.at[pl.ds(0, ZCH)],
                                acc.at[pl.ds(ch * ZCH, ZCH)])

        plsc.subcore_barrier()

        def body(kk, carry):
            g = kk * NW + wid

            @pl.when(g < NG)
            def _():
                off = g * GROUP
                pltpu.sync_copy(r_hbm.at[pl.ds(off, GROUP)], ridx)
                pltpu.sync_copy(ue_hbm.at[pl.ds(off, GROUP)], buf)
                pltpu.sync_copy(buf, acc.at[ridx], add=True)

            return carry

        lax.fori_loop(0, KMAX, body, 0)
        plsc.subcore_barrier()

        for q in range(ZROUNDS):
            ch = sid + q * NS

            @pl.when(ch < NCH)
            def _():
                off = ch * ZCH
                pltpu.sync_copy(acc.at[pl.ds(off, ZCH)], buf.at[pl.ds(0, ZCH)])
                pltpu.sync_copy(buf.at[pl.ds(0, ZCH)],
                                out_hbm.at[cid, pl.ds(off, ZCH)])

    return k(ue, receivers)


def _edge_body(gs, gr, ea, w0s, w0r, w0e, w1, w2, w3,
               b0, b1, b2, b3, g, beta, ue_ref, uep_ref):
    f32 = jnp.float32
    ea_x = ea[...]
    x = (jnp.dot(gs[...], w0s[...], preferred_element_type=f32)
         + jnp.dot(gr[...], w0r[...], preferred_element_type=f32)
         + jnp.dot(ea_x, w0e[...], preferred_element_type=f32) + b0[...])
    x = jnp.maximum(x, 0.0)
    x = jnp.maximum(jnp.dot(x, w1[...], preferred_element_type=f32) + b1[...], 0.0)
    x = jnp.maximum(jnp.dot(x, w2[...], preferred_element_type=f32) + b2[...], 0.0)
    h = jnp.dot(x, w3[...], preferred_element_type=f32) + b3[...]
    mu = jnp.mean(h, axis=-1, keepdims=True)
    d = h - mu
    var = jnp.mean(d * d, axis=-1, keepdims=True)
    u = d * lax.rsqrt(var + 1e-5) * g[...] + beta[...]
    ue_ref[...] = u
    uep_ref[...] = u + ea_x


def _edge_mlp(gs, gr, ea, w0s, w0r, w0e, w1, w2, w3, b0, b1, b2, b3, g, beta):
    E, Hd = ea.shape
    BE = 512
    assert E % BE == 0
    row = lambda i: (i, 0)
    const = lambda i: (0, 0)
    mspec = pl.BlockSpec((BE, Hd), row)
    wspec = pl.BlockSpec((Hd, Hd), const)
    vspec = pl.BlockSpec((1, Hd), const)
    return pl.pallas_call(
        _edge_body,
        grid=(E // BE,),
        in_specs=[mspec, mspec, mspec, wspec, wspec, wspec, wspec, wspec, wspec,
                  vspec, vspec, vspec, vspec, vspec, vspec],
        out_specs=(mspec, mspec),
        out_shape=(jax.ShapeDtypeStruct((E, Hd), jnp.float32),
                   jax.ShapeDtypeStruct((E, Hd), jnp.float32)),
        compiler_params=pltpu.CompilerParams(
            dimension_semantics=("arbitrary",)),
    )(gs, gr, ea, w0s, w0r, w0e, w1, w2, w3,
      b0.reshape(1, Hd), b1.reshape(1, Hd), b2.reshape(1, Hd), b3.reshape(1, Hd),
      g.reshape(1, Hd), beta.reshape(1, Hd))


def _node_body(na, parts, w0n, w0a, w1, w2, w3, b0, b1, b2, b3, g, beta, out_ref):
    f32 = jnp.float32
    na_x = na[...]
    agg = parts[0] + parts[1]
    x = (jnp.dot(na_x, w0n[...], preferred_element_type=f32)
         + jnp.dot(agg, w0a[...], preferred_element_type=f32) + b0[...])
    x = jnp.maximum(x, 0.0)
    x = jnp.maximum(jnp.dot(x, w1[...], preferred_element_type=f32) + b1[...], 0.0)
    x = jnp.maximum(jnp.dot(x, w2[...], preferred_element_type=f32) + b2[...], 0.0)
    h = jnp.dot(x, w3[...], preferred_element_type=f32) + b3[...]
    mu = jnp.mean(h, axis=-1, keepdims=True)
    d = h - mu
    var = jnp.mean(d * d, axis=-1, keepdims=True)
    out_ref[...] = d * lax.rsqrt(var + 1e-5) * g[...] + beta[...] + na_x


def _node_mlp(na, parts, w0n, w0a, w1, w2, w3, b0, b1, b2, b3, g, beta):
    N, Hd = na.shape
    BN = 1000
    assert N % BN == 0
    mspec = pl.BlockSpec((BN, Hd), lambda i: (i, 0))
    pspec = pl.BlockSpec((2, BN, Hd), lambda i: (0, i, 0))
    const = lambda i: (0, 0)
    wspec = pl.BlockSpec((Hd, Hd), const)
    vspec = pl.BlockSpec((1, Hd), const)
    return pl.pallas_call(
        _node_body,
        grid=(N // BN,),
        in_specs=[mspec, pspec, wspec, wspec, wspec, wspec, wspec,
                  vspec, vspec, vspec, vspec, vspec, vspec],
        out_specs=mspec,
        out_shape=jax.ShapeDtypeStruct((N, Hd), jnp.float32),
        compiler_params=pltpu.CompilerParams(
            dimension_semantics=("arbitrary",)),
    )(na, parts, w0n, w0a, w1, w2, w3,
      b0.reshape(1, Hd), b1.reshape(1, Hd), b2.reshape(1, Hd), b3.reshape(1, Hd),
      g.reshape(1, Hd), beta.reshape(1, Hd))


def kernel(node_attr, edge_attr, edge_index, num_nodes,
           eb_W0, eb_b0, eb_W1, eb_b1, eb_W2, eb_b2, eb_W3, eb_b3, eb_g, eb_beta,
           nb_W0, nb_b0, nb_W1, nb_b1, nb_W2, nb_b2, nb_W3, nb_b3, nb_g, nb_beta):
    N, Hd = node_attr.shape
    senders = edge_index[0]
    receivers = edge_index[1]

    gs, gr = _sc_gather(node_attr, senders, receivers)
    ue, uep = _edge_mlp(
        gs, gr, edge_attr,
        eb_W0[:Hd], eb_W0[Hd:2 * Hd], eb_W0[2 * Hd:],
        eb_W1, eb_W2, eb_W3, eb_b0, eb_b1, eb_b2, eb_b3, eb_g, eb_beta)
    parts = _sc_scatter(ue, receivers, N)
    un = _node_mlp(
        node_attr, parts,
        nb_W0[:Hd], nb_W0[Hd:],
        nb_W1, nb_W2, nb_W3, nb_b0, nb_b1, nb_b2, nb_b3, nb_g, nb_beta)
    return (un, uep)
